# baseline (speedup 1.0000x reference)
import jax
import jax.numpy as jnp
from jax import lax
from jax.experimental import pallas as pl
from jax.experimental.pallas import tpu as pltpu
from jax.experimental.pallas import tpu_sc as plsc

N = 10000
F = 128
E = 320000
TWO_N = 2 * N

NC = 2
NS = 16
NW = NC * NS

NDUM = 8

E_PER_W = E // NW
CH_A = 2000
N_CH_A = E_PER_W // CH_A
RSTRIDE = 10256


def _sc_prep_body(src_hbm, dst_hbm, typ_hbm, degp_out, sp0, dp0, sp1, dp1, cnt_out,
                  src_v, dst_v, typ_v, sb0, db0, sb1, db1, cnt_v, deg_v):
    c = lax.axis_index("c")
    s = lax.axis_index("s")
    wid = s * NC + c

    def zero_body(j, carry):
        deg_v[pl.ds(pl.multiple_of(j * 16, 16), 16)] = jnp.zeros((16,), jnp.float32)
        return carry

    lax.fori_loop(0, TWO_N // 16, zero_body, 0)

    ones16 = jnp.ones((16,), jnp.float32)
    cur0, cur1 = jnp.int32(0), jnp.int32(0)
    for c5 in range(N_CH_A):
        base = pl.multiple_of(wid * E_PER_W + c5 * CH_A, 8)
        pltpu.sync_copy(src_hbm.at[pl.ds(base, CH_A)], src_v)
        pltpu.sync_copy(dst_hbm.at[pl.ds(base, CH_A)], dst_v)
        pltpu.sync_copy(typ_hbm.at[pl.ds(base, CH_A)], typ_v)

        def body(j, carry):
            c0, c1 = carry
            sl = pl.ds(pl.multiple_of(j * 16, 16), 16)
            s16 = src_v[sl]
            d16 = dst_v[sl]
            t16 = typ_v[sl]
            m0 = t16 == 0
            plsc.store_compressed(sb0.at[pl.ds(c0, 16)], s16, mask=m0)
            plsc.store_compressed(db0.at[pl.ds(c0, 16)], d16, mask=m0)
            m1 = jnp.logical_not(m0)
            plsc.store_compressed(sb1.at[pl.ds(c1, 16)], s16, mask=m1)
            plsc.store_compressed(db1.at[pl.ds(c1, 16)], d16, mask=m1)
            plsc.addupdate_scatter(deg_v, [d16 * 2 + t16], ones16)
            n0 = jnp.sum(jnp.where(m0, 1, 0))
            return (c0 + n0, c1 + (16 - n0))

        cur0, cur1 = lax.fori_loop(0, CH_A // 16, body, (cur0, cur1))

    sfill = jnp.zeros((16,), jnp.int32)
    dfill = jnp.full((16,), N, jnp.int32)
    ftrue = sfill == 0

    def pad_fill(sb, db, cur):
        bound = ((cur + 255) >> 8) << 8
        nf = (bound - cur + 15) >> 4

        def fbody(i, carry):
            plsc.store_compressed(sb.at[pl.ds(cur + i * 16, 16)], sfill, mask=ftrue)
            plsc.store_compressed(db.at[pl.ds(cur + i * 16, 16)], dfill, mask=ftrue)
            return carry

        lax.fori_loop(0, nf, fbody, 0)
        return bound >> 8

    k0 = pad_fill(sb0, db0, cur0)
    k1 = pad_fill(sb1, db1, cur1)
    lane = lax.iota(jnp.int32, 16)
    cnt_v[pl.ds(0, 16)] = jnp.where(lane == 0, k0, jnp.where(lane == 1, k1, 0))

    rb = pl.multiple_of(wid * RSTRIDE, 8)
    pltpu.sync_copy(sb0, sp0.at[pl.ds(rb, RSTRIDE)])
    pltpu.sync_copy(db0, dp0.at[pl.ds(rb, RSTRIDE)])
    pltpu.sync_copy(sb1, sp1.at[pl.ds(rb, RSTRIDE)])
    pltpu.sync_copy(db1, dp1.at[pl.ds(rb, RSTRIDE)])
    pltpu.sync_copy(cnt_v, cnt_out.at[wid])
    pltpu.sync_copy(deg_v, degp_out.at[wid])


@jax.jit
def _sc_prep(src, dst, typ):
    mesh = plsc.VectorSubcoreMesh(core_axis_name="c", subcore_axis_name="s")
    fn = pl.kernel(
        _sc_prep_body,
        mesh=mesh,
        compiler_params=pltpu.CompilerParams(needs_layout_passes=False),
        out_type=[
            jax.ShapeDtypeStruct((NW, TWO_N), jnp.float32),
            jax.ShapeDtypeStruct((NW * RSTRIDE,), jnp.int32),
            jax.ShapeDtypeStruct((NW * RSTRIDE,), jnp.int32),
            jax.ShapeDtypeStruct((NW * RSTRIDE,), jnp.int32),
            jax.ShapeDtypeStruct((NW * RSTRIDE,), jnp.int32),
            jax.ShapeDtypeStruct((NW, 16), jnp.int32),
        ],
        scratch_types=[
            pltpu.VMEM((CH_A,), jnp.int32),
            pltpu.VMEM((CH_A,), jnp.int32),
            pltpu.VMEM((CH_A,), jnp.int32),
            pltpu.VMEM((RSTRIDE,), jnp.int32),
            pltpu.VMEM((RSTRIDE,), jnp.int32),
            pltpu.VMEM((RSTRIDE,), jnp.int32),
            pltpu.VMEM((RSTRIDE,), jnp.int32),
            pltpu.VMEM((16,), jnp.int32),
            pltpu.VMEM((TWO_N,), jnp.float32),
        ],
    )
    return fn(src, dst, typ)


BN = 1000


def _tc_prep_body(degp_ref, x_ref, dinv_ref, y0_ref, y1_ref):
    deg = jnp.sum(degp_ref[...], axis=0) + 1.0
    dinv = lax.rsqrt(deg)
    dinv_ref[...] = dinv
    xb = x_ref[...]
    y0_ref[...] = dinv[:, 0:1] * xb
    y1_ref[...] = dinv[:, 1:2] * xb


@jax.jit
def _tc_prep(degp, x):
    return pl.pallas_call(
        _tc_prep_body,
        grid=(N // BN,),
        in_specs=[
            pl.BlockSpec((NW, BN, 2), lambda i: (0, i, 0)),
            pl.BlockSpec((BN, F), lambda i: (i, 0)),
        ],
        out_specs=[
            pl.BlockSpec((BN, 2), lambda i: (i, 0)),
            pl.BlockSpec((BN, F), lambda i: (i, 0)),
            pl.BlockSpec((BN, F), lambda i: (i, 0)),
        ],
        out_shape=[
            jax.ShapeDtypeStruct((N, 2), jnp.float32),
            jax.ShapeDtypeStruct((N, F), jnp.float32),
            jax.ShapeDtypeStruct((N, F), jnp.float32),
        ],
    )(degp, x)


CH_C = 32
NSLOT = 8
ZROWS = 624
ZTAIL = N - NS * ZROWS
ZCH = 48


def _sc_agg_body(y0, y1, sp0, dp0, sp1, dp1, cnt, qout, *scr):
    c = lax.axis_index("c")
    s = lax.axis_index("s")
    sixs = scr[0:NSLOT]
    dixs = scr[NSLOT:2 * NSLOT]
    rows = scr[2 * NSLOT:3 * NSLOT]
    cnt_v, zbuf, q_sh = scr[3 * NSLOT:3 * NSLOT + 3]
    semg = scr[3 * NSLOT + 3:4 * NSLOT + 3]
    sems = scr[4 * NSLOT + 3:5 * NSLOT + 3]
    semi = scr[5 * NSLOT + 3:6 * NSLOT + 3]

    PROBE_SKIP_ZW = False
    z16 = jnp.zeros((16,), jnp.float32)

    def zb_body(r, carry):
        for k in range(F // 16):
            zbuf[r, pl.ds(k * 16, 16)] = z16
        return carry

    lax.fori_loop(0, ZCH, zb_body, 0)
    for kk in range(ZROWS // ZCH if not PROBE_SKIP_ZW else 0):
        zsl = pl.ds(pl.multiple_of(s * ZROWS + kk * ZCH, 8), ZCH)
        pltpu.sync_copy(zbuf, q_sh.at[zsl])

    @pl.when(s < ZTAIL // 8)
    def _():
        tsl = pl.ds(pl.multiple_of(NS * ZROWS + s * 8, 8), 8)
        pltpu.sync_copy(zbuf.at[pl.ds(0, 8)], q_sh.at[tsl])

    @pl.when(s == 2)
    def _():
        dsl = pl.ds(pl.multiple_of(N, 8), NDUM)
        pltpu.sync_copy(zbuf.at[pl.ds(0, NDUM)], q_sh.at[dsl])

    plsc.subcore_barrier()

    def region_loop(ysel, ssel, dsel, tt):
        for rr in range(2):
            r = s * 2 + rr
            pltpu.sync_copy(cnt.at[r], cnt_v)
            kmax2 = cnt_v[pl.ds(0, 16)][tt]
            rbase = r * RSTRIDE

            @pl.when(kmax2 > 0)
            def _():
                nch = kmax2 * NSLOT

                def idx_start(k, slot):
                    bp = pl.multiple_of(rbase + k * CH_C, 8)
                    pltpu.async_copy(ssel.at[pl.ds(bp, CH_C)], sixs[slot],
                                     semi[slot])
                    pltpu.async_copy(dsel.at[pl.ds(bp, CH_C)], dixs[slot],
                                     semi[slot])

                def idx_wait(k, slot):
                    bp = pl.multiple_of(rbase + k * CH_C, 8)
                    pltpu.make_async_copy(ssel.at[pl.ds(bp, CH_C)],
                                          sixs[slot], semi[slot]).wait()
                    pltpu.make_async_copy(dsel.at[pl.ds(bp, CH_C)],
                                          dixs[slot], semi[slot]).wait()

                for p in range(NSLOT - 1):
                    idx_start(p, p)
                    idx_wait(p, p)
                    pltpu.async_copy(ysel.at[sixs[p]], rows[p], semg[p])

                @pl.when(NSLOT - 1 < nch)
                def _():
                    idx_start(NSLOT - 1, NSLOT - 1)

                def group_body(k2, carry):
                    for b in range(NSLOT):
                        o = (b + NSLOT - 1) % NSLOT
                        k = k2 * NSLOT + b

                        pltpu.make_async_copy(
                            ysel.at[sixs[b]], rows[b], semg[b]).wait()
                        pltpu.async_copy(
                            rows[b], q_sh.at[dixs[b]], sems[b], add=True)

                        @pl.when(k >= 1)
                        def _():
                            pltpu.make_async_copy(
                                rows[o], q_sh.at[dixs[o]], sems[o]).wait()

                        @pl.when(k + NSLOT - 1 < nch)
                        def _():
                            idx_wait(k + NSLOT - 1, o)
                            pltpu.async_copy(ysel.at[sixs[o]], rows[o], semg[o])

                        @pl.when(k + NSLOT < nch)
                        def _():
                            idx_start(k + NSLOT, b)
                    return carry

                lax.fori_loop(0, kmax2, group_body, 0)
                pltpu.make_async_copy(
                    rows[NSLOT - 1], q_sh.at[dixs[NSLOT - 1]],
                    sems[NSLOT - 1]).wait()

    PROBE_SKIP_EDGES = False
    if not PROBE_SKIP_EDGES:
        @pl.when(c == 0)
        def _():
            region_loop(y0, sp0, dp0, 0)

        @pl.when(c == 1)
        def _():
            region_loop(y1, sp1, dp1, 1)

    plsc.subcore_barrier()
    for kk in range(ZROWS // ZCH if not PROBE_SKIP_ZW else 1):
        zsl = pl.ds(pl.multiple_of(s * ZROWS + kk * ZCH, 8), ZCH)
        pltpu.sync_copy(q_sh.at[zsl], zbuf)
        pltpu.sync_copy(zbuf, qout.at[c].at[zsl])

    @pl.when(s < ZTAIL // 8)
    def _():
        tsl = pl.ds(pl.multiple_of(NS * ZROWS + s * 8, 8), 8)
        pltpu.sync_copy(q_sh.at[tsl], zbuf.at[pl.ds(0, 8)])
        pltpu.sync_copy(zbuf.at[pl.ds(0, 8)], qout.at[c].at[tsl])


@jax.jit
def _sc_agg(y0, y1, sp0, dp0, sp1, dp1, cnt):
    mesh = plsc.VectorSubcoreMesh(core_axis_name="c", subcore_axis_name="s")
    fn = pl.kernel(
        _sc_agg_body,
        mesh=mesh,
        compiler_params=pltpu.CompilerParams(needs_layout_passes=False),
        out_type=[jax.ShapeDtypeStruct((NC, N, F), jnp.float32)],
        scratch_types=(
            [pltpu.VMEM((CH_C,), jnp.int32)] * (2 * NSLOT)
            + [pltpu.VMEM((CH_C, F), jnp.float32)] * NSLOT
            + [
                pltpu.VMEM((16,), jnp.int32),
                pltpu.VMEM((ZCH, F), jnp.float32),
                pltpu.VMEM_SHARED((N + NDUM, F), jnp.float32),
            ]
            + [pltpu.SemaphoreType.DMA] * (3 * NSLOT)
        ),
    )
    return fn(y0, y1, sp0, dp0, sp1, dp1, cnt)


def _tc_final_body(x_ref, qn_ref, qd_ref, dinv_ref,
                   wn, bn, wd, bd, wsl, bsl, wg, bg, wdr, bdr, out_ref):
    xb = x_ref[...]
    d0 = dinv_ref[:, 0:1]
    d1 = dinv_ref[:, 1:2]
    pre_n = d0 * (qn_ref[0] + d0 * xb)
    pre_d = d1 * (qd_ref[0] + d1 * xb)

    def mm(a, w):
        return jnp.dot(a, w[...], preferred_element_type=jnp.float32)

    xn = mm(pre_n, wn) + bn[...]
    xd = mm(pre_d, wd) + bd[...]
    xx = mm(xb, wsl) + bsl[...]
    wgr = wg[...]
    z = (jnp.dot(xx, wgr[0:F, :], preferred_element_type=jnp.float32)
         + jnp.dot(xn, wgr[F:2 * F, :], preferred_element_type=jnp.float32)
         + jnp.dot(xd, wgr[2 * F:3 * F, :], preferred_element_type=jnp.float32)
         + bg[...])
    m = jnp.max(z, axis=-1, keepdims=True)
    ez = jnp.exp(z - m)
    sm = ez / jnp.sum(ez, axis=-1, keepdims=True)
    rr = lax.broadcasted_iota(jnp.int32, (F, F), 0)
    cc = lax.broadcasted_iota(jnp.int32, (F, F), 1)
    tri = (rr <= cc).astype(jnp.float32)
    gat = jnp.dot(sm, tri, preferred_element_type=jnp.float32)
    xdr = mm(pre_d, wdr) + bdr[...]
    out_ref[...] = xdr * gat + xx + xn


@jax.jit
def _tc_final(x, q2, dinv, wn, bn, wd, bd, wsl, bsl, wg, bg, wdr, bdr):
    wspec = lambda shape: pl.BlockSpec(shape, lambda i: tuple(0 for _ in shape))
    return pl.pallas_call(
        _tc_final_body,
        grid=(N // BN,),
        in_specs=[
            pl.BlockSpec((BN, F), lambda i: (i, 0)),
            pl.BlockSpec((1, BN, F), lambda i: (0, i, 0)),
            pl.BlockSpec((1, BN, F), lambda i: (1, i, 0)),
            pl.BlockSpec((BN, 2), lambda i: (i, 0)),
            wspec((F, F)), wspec((1, F)),
            wspec((F, F)), wspec((1, F)),
            wspec((F, F)), wspec((1, F)),
            wspec((3 * F, F)), wspec((1, F)),
            wspec((F, F)), wspec((1, F)),
        ],
        out_specs=pl.BlockSpec((BN, F), lambda i: (i, 0)),
        out_shape=jax.ShapeDtypeStruct((N, F), jnp.float32),
    )(x, q2, q2, dinv, wn, bn, wd, bd, wsl, bsl, wg, bg, wdr, bdr)


def kernel(x, edge_index, edge_type, W_n, b_n, W_d, b_d, W_sl, b_sl, W_gat, b_gat):
    src = edge_index[0].astype(jnp.int32)
    dst = edge_index[1].astype(jnp.int32)
    typ = edge_type.astype(jnp.int32)

    PROBE_SKIP_PREP = False
    if PROBE_SKIP_PREP:
        degp = jnp.zeros((NW, TWO_N), jnp.float32)
        sp0 = jnp.zeros((NW * RSTRIDE,), jnp.int32)
        dp0 = jnp.zeros((NW * RSTRIDE,), jnp.int32)
        sp1 = jnp.zeros((NW * RSTRIDE,), jnp.int32)
        dp1 = jnp.zeros((NW * RSTRIDE,), jnp.int32)
        cnt = jnp.zeros((NW, 16), jnp.int32)
    else:
        degp, sp0, dp0, sp1, dp1, cnt = _sc_prep(src, dst, typ)
    dinv, y0, y1 = _tc_prep(degp.reshape(NW, N, 2), x)
    (q2,) = _sc_agg(y0, y1, sp0, dp0, sp1, dp1, cnt)
    out = _tc_final(
        x, q2, dinv,
        W_n, b_n.reshape(1, F), W_d, b_d.reshape(1, F),
        W_sl, b_sl.reshape(1, F), W_gat, b_gat.reshape(1, F),
        W_d[:, ::-1], b_d[::-1].reshape(1, F),
    )
    return out


# speedup vs baseline: 1.0272x; 1.0272x over previous
import jax
import jax.numpy as jnp
from jax import lax
from jax.experimental import pallas as pl
from jax.experimental.pallas import tpu as pltpu
from jax.experimental.pallas import tpu_sc as plsc

N = 10000
F = 128
E = 320000
TWO_N = 2 * N

NC = 2
NS = 16
NW = NC * NS

NDUM = 8

E_PER_W = E // NW
CH_A = 2000
N_CH_A = E_PER_W // CH_A
RSTRIDE = 10256


def _sc_prep_body(src_hbm, dst_hbm, typ_hbm, degp_out, sp0, dp0, sp1, dp1, cnt_out,
                  src_v, dst_v, typ_v, sb0, db0, sb1, db1, cnt_v, deg_v):
    c = lax.axis_index("c")
    s = lax.axis_index("s")
    wid = s * NC + c

    def zero_body(j, carry):
        deg_v[pl.ds(pl.multiple_of(j * 16, 16), 16)] = jnp.zeros((16,), jnp.float32)
        return carry

    lax.fori_loop(0, TWO_N // 16, zero_body, 0)

    ones16 = jnp.ones((16,), jnp.float32)
    cur0, cur1 = jnp.int32(0), jnp.int32(0)
    for c5 in range(N_CH_A):
        base = pl.multiple_of(wid * E_PER_W + c5 * CH_A, 8)
        pltpu.sync_copy(src_hbm.at[pl.ds(base, CH_A)], src_v)
        pltpu.sync_copy(dst_hbm.at[pl.ds(base, CH_A)], dst_v)
        pltpu.sync_copy(typ_hbm.at[pl.ds(base, CH_A)], typ_v)

        def body(j, carry):
            c0, c1 = carry
            sl = pl.ds(pl.multiple_of(j * 16, 16), 16)
            s16 = src_v[sl]
            d16 = dst_v[sl]
            t16 = typ_v[sl]
            m0 = t16 == 0
            plsc.store_compressed(sb0.at[pl.ds(c0, 16)], s16, mask=m0)
            plsc.store_compressed(db0.at[pl.ds(c0, 16)], d16, mask=m0)
            m1 = jnp.logical_not(m0)
            plsc.store_compressed(sb1.at[pl.ds(c1, 16)], s16, mask=m1)
            plsc.store_compressed(db1.at[pl.ds(c1, 16)], d16, mask=m1)
            plsc.addupdate_scatter(deg_v, [d16 * 2 + t16], ones16)
            n0 = jnp.sum(jnp.where(m0, 1, 0))
            return (c0 + n0, c1 + (16 - n0))

        cur0, cur1 = lax.fori_loop(0, CH_A // 16, body, (cur0, cur1))

    sfill = jnp.zeros((16,), jnp.int32)
    dfill = jnp.full((16,), N, jnp.int32)
    ftrue = sfill == 0

    def pad_fill(sb, db, cur):
        bound = ((cur + 255) >> 8) << 8
        nf = (bound - cur + 15) >> 4

        def fbody(i, carry):
            plsc.store_compressed(sb.at[pl.ds(cur + i * 16, 16)], sfill, mask=ftrue)
            plsc.store_compressed(db.at[pl.ds(cur + i * 16, 16)], dfill, mask=ftrue)
            return carry

        lax.fori_loop(0, nf, fbody, 0)
        return bound >> 8

    k0 = pad_fill(sb0, db0, cur0)
    k1 = pad_fill(sb1, db1, cur1)
    lane = lax.iota(jnp.int32, 16)
    cnt_v[pl.ds(0, 16)] = jnp.where(lane == 0, k0, jnp.where(lane == 1, k1, 0))

    rb = pl.multiple_of(wid * RSTRIDE, 8)
    pltpu.sync_copy(sb0, sp0.at[pl.ds(rb, RSTRIDE)])
    pltpu.sync_copy(db0, dp0.at[pl.ds(rb, RSTRIDE)])
    pltpu.sync_copy(sb1, sp1.at[pl.ds(rb, RSTRIDE)])
    pltpu.sync_copy(db1, dp1.at[pl.ds(rb, RSTRIDE)])
    pltpu.sync_copy(cnt_v, cnt_out.at[wid])
    pltpu.sync_copy(deg_v, degp_out.at[wid])


@jax.jit
def _sc_prep(src, dst, typ):
    mesh = plsc.VectorSubcoreMesh(core_axis_name="c", subcore_axis_name="s")
    fn = pl.kernel(
        _sc_prep_body,
        mesh=mesh,
        compiler_params=pltpu.CompilerParams(needs_layout_passes=False),
        out_type=[
            jax.ShapeDtypeStruct((NW, TWO_N), jnp.float32),
            jax.ShapeDtypeStruct((NW * RSTRIDE,), jnp.int32),
            jax.ShapeDtypeStruct((NW * RSTRIDE,), jnp.int32),
            jax.ShapeDtypeStruct((NW * RSTRIDE,), jnp.int32),
            jax.ShapeDtypeStruct((NW * RSTRIDE,), jnp.int32),
            jax.ShapeDtypeStruct((NW, 16), jnp.int32),
        ],
        scratch_types=[
            pltpu.VMEM((CH_A,), jnp.int32),
            pltpu.VMEM((CH_A,), jnp.int32),
            pltpu.VMEM((CH_A,), jnp.int32),
            pltpu.VMEM((RSTRIDE,), jnp.int32),
            pltpu.VMEM((RSTRIDE,), jnp.int32),
            pltpu.VMEM((RSTRIDE,), jnp.int32),
            pltpu.VMEM((RSTRIDE,), jnp.int32),
            pltpu.VMEM((16,), jnp.int32),
            pltpu.VMEM((TWO_N,), jnp.float32),
        ],
    )
    return fn(src, dst, typ)


BN = 1000


def _tc_prep_body(degp_ref, x_ref, dinv_ref, y0_ref, y1_ref):
    deg = jnp.sum(degp_ref[...], axis=0) + 1.0
    dinv = lax.rsqrt(deg)
    dinv_ref[...] = dinv
    xb = x_ref[...]
    y0_ref[...] = dinv[:, 0:1] * xb
    y1_ref[...] = dinv[:, 1:2] * xb


@jax.jit
def _tc_prep(degp, x):
    return pl.pallas_call(
        _tc_prep_body,
        grid=(N // BN,),
        in_specs=[
            pl.BlockSpec((NW, BN, 2), lambda i: (0, i, 0)),
            pl.BlockSpec((BN, F), lambda i: (i, 0)),
        ],
        out_specs=[
            pl.BlockSpec((BN, 2), lambda i: (i, 0)),
            pl.BlockSpec((BN, F), lambda i: (i, 0)),
            pl.BlockSpec((BN, F), lambda i: (i, 0)),
        ],
        out_shape=[
            jax.ShapeDtypeStruct((N, 2), jnp.float32),
            jax.ShapeDtypeStruct((N, F), jnp.float32),
            jax.ShapeDtypeStruct((N, F), jnp.float32),
        ],
    )(degp, x)


CH_C = 128
NSLOT = 2
ZROWS = 624
ZTAIL = N - NS * ZROWS
ZCH = 48


def _sc_agg_body(y0, y1, sp0, dp0, sp1, dp1, cnt, qout, *scr):
    c = lax.axis_index("c")
    s = lax.axis_index("s")
    sixs = scr[0:NSLOT]
    dixs = scr[NSLOT:2 * NSLOT]
    rows = scr[2 * NSLOT:3 * NSLOT]
    cnt_v, zbuf, q_sh = scr[3 * NSLOT:3 * NSLOT + 3]
    semg = scr[3 * NSLOT + 3:4 * NSLOT + 3]
    sems = scr[4 * NSLOT + 3:5 * NSLOT + 3]
    semi = scr[5 * NSLOT + 3:6 * NSLOT + 3]

    PROBE_SKIP_ZW = False
    z16 = jnp.zeros((16,), jnp.float32)

    def zb_body(r, carry):
        for k in range(F // 16):
            zbuf[r, pl.ds(k * 16, 16)] = z16
        return carry

    lax.fori_loop(0, ZCH, zb_body, 0)
    for kk in range(ZROWS // ZCH if not PROBE_SKIP_ZW else 0):
        zsl = pl.ds(pl.multiple_of(s * ZROWS + kk * ZCH, 8), ZCH)
        pltpu.sync_copy(zbuf, q_sh.at[zsl])

    @pl.when(s < ZTAIL // 8)
    def _():
        tsl = pl.ds(pl.multiple_of(NS * ZROWS + s * 8, 8), 8)
        pltpu.sync_copy(zbuf.at[pl.ds(0, 8)], q_sh.at[tsl])

    @pl.when(s == 2)
    def _():
        dsl = pl.ds(pl.multiple_of(N, 8), NDUM)
        pltpu.sync_copy(zbuf.at[pl.ds(0, NDUM)], q_sh.at[dsl])

    plsc.subcore_barrier()

    def region_loop(ysel, ssel, dsel, tt):
        for rr in range(2):
            r = s * 2 + rr
            pltpu.sync_copy(cnt.at[r], cnt_v)
            kmax2 = cnt_v[pl.ds(0, 16)][tt]
            rbase = r * RSTRIDE

            @pl.when(kmax2 > 0)
            def _():
                nch = kmax2 * NSLOT

                def idx_start(k, slot):
                    bp = pl.multiple_of(rbase + k * CH_C, 8)
                    pltpu.async_copy(ssel.at[pl.ds(bp, CH_C)], sixs[slot],
                                     semi[slot])
                    pltpu.async_copy(dsel.at[pl.ds(bp, CH_C)], dixs[slot],
                                     semi[slot])

                def idx_wait(k, slot):
                    bp = pl.multiple_of(rbase + k * CH_C, 8)
                    pltpu.make_async_copy(ssel.at[pl.ds(bp, CH_C)],
                                          sixs[slot], semi[slot]).wait()
                    pltpu.make_async_copy(dsel.at[pl.ds(bp, CH_C)],
                                          dixs[slot], semi[slot]).wait()

                for p in range(NSLOT - 1):
                    idx_start(p, p)
                    idx_wait(p, p)
                    pltpu.async_copy(ysel.at[sixs[p]], rows[p], semg[p])

                @pl.when(NSLOT - 1 < nch)
                def _():
                    idx_start(NSLOT - 1, NSLOT - 1)

                def group_body(k2, carry):
                    for b in range(NSLOT):
                        o = (b + NSLOT - 1) % NSLOT
                        k = k2 * NSLOT + b

                        pltpu.make_async_copy(
                            ysel.at[sixs[b]], rows[b], semg[b]).wait()
                        pltpu.async_copy(
                            rows[b], q_sh.at[dixs[b]], sems[b], add=True)

                        @pl.when(k >= 1)
                        def _():
                            pltpu.make_async_copy(
                                rows[o], q_sh.at[dixs[o]], sems[o]).wait()

                        @pl.when(k + NSLOT - 1 < nch)
                        def _():
                            idx_wait(k + NSLOT - 1, o)
                            pltpu.async_copy(ysel.at[sixs[o]], rows[o], semg[o])

                        @pl.when(k + NSLOT < nch)
                        def _():
                            idx_start(k + NSLOT, b)
                    return carry

                lax.fori_loop(0, kmax2, group_body, 0)
                pltpu.make_async_copy(
                    rows[NSLOT - 1], q_sh.at[dixs[NSLOT - 1]],
                    sems[NSLOT - 1]).wait()

    PROBE_SKIP_EDGES = False
    if not PROBE_SKIP_EDGES:
        @pl.when(c == 0)
        def _():
            region_loop(y0, sp0, dp0, 0)

        @pl.when(c == 1)
        def _():
            region_loop(y1, sp1, dp1, 1)

    plsc.subcore_barrier()
    for kk in range(ZROWS // ZCH if not PROBE_SKIP_ZW else 1):
        zsl = pl.ds(pl.multiple_of(s * ZROWS + kk * ZCH, 8), ZCH)
        pltpu.sync_copy(q_sh.at[zsl], zbuf)
        pltpu.sync_copy(zbuf, qout.at[c].at[zsl])

    @pl.when(s < ZTAIL // 8)
    def _():
        tsl = pl.ds(pl.multiple_of(NS * ZROWS + s * 8, 8), 8)
        pltpu.sync_copy(q_sh.at[tsl], zbuf.at[pl.ds(0, 8)])
        pltpu.sync_copy(zbuf.at[pl.ds(0, 8)], qout.at[c].at[tsl])


@jax.jit
def _sc_agg(y0, y1, sp0, dp0, sp1, dp1, cnt):
    mesh = plsc.VectorSubcoreMesh(core_axis_name="c", subcore_axis_name="s")
    fn = pl.kernel(
        _sc_agg_body,
        mesh=mesh,
        compiler_params=pltpu.CompilerParams(needs_layout_passes=False),
        out_type=[jax.ShapeDtypeStruct((NC, N, F), jnp.float32)],
        scratch_types=(
            [pltpu.VMEM((CH_C,), jnp.int32)] * (2 * NSLOT)
            + [pltpu.VMEM((CH_C, F), jnp.float32)] * NSLOT
            + [
                pltpu.VMEM((16,), jnp.int32),
                pltpu.VMEM((ZCH, F), jnp.float32),
                pltpu.VMEM_SHARED((N + NDUM, F), jnp.float32),
            ]
            + [pltpu.SemaphoreType.DMA] * (3 * NSLOT)
        ),
    )
    return fn(y0, y1, sp0, dp0, sp1, dp1, cnt)


def _tc_final_body(x_ref, qn_ref, qd_ref, dinv_ref,
                   wn, bn, wd, bd, wsl, bsl, wg, bg, wdr, bdr, out_ref):
    xb = x_ref[...]
    d0 = dinv_ref[:, 0:1]
    d1 = dinv_ref[:, 1:2]
    pre_n = d0 * (qn_ref[0] + d0 * xb)
    pre_d = d1 * (qd_ref[0] + d1 * xb)

    def mm(a, w):
        return jnp.dot(a, w[...], preferred_element_type=jnp.float32)

    xn = mm(pre_n, wn) + bn[...]
    xd = mm(pre_d, wd) + bd[...]
    xx = mm(xb, wsl) + bsl[...]
    wgr = wg[...]
    z = (jnp.dot(xx, wgr[0:F, :], preferred_element_type=jnp.float32)
         + jnp.dot(xn, wgr[F:2 * F, :], preferred_element_type=jnp.float32)
         + jnp.dot(xd, wgr[2 * F:3 * F, :], preferred_element_type=jnp.float32)
         + bg[...])
    m = jnp.max(z, axis=-1, keepdims=True)
    ez = jnp.exp(z - m)
    sm = ez / jnp.sum(ez, axis=-1, keepdims=True)
    rr = lax.broadcasted_iota(jnp.int32, (F, F), 0)
    cc = lax.broadcasted_iota(jnp.int32, (F, F), 1)
    tri = (rr <= cc).astype(jnp.float32)
    gat = jnp.dot(sm, tri, preferred_element_type=jnp.float32)
    xdr = mm(pre_d, wdr) + bdr[...]
    out_ref[...] = xdr * gat + xx + xn


@jax.jit
def _tc_final(x, q2, dinv, wn, bn, wd, bd, wsl, bsl, wg, bg, wdr, bdr):
    wspec = lambda shape: pl.BlockSpec(shape, lambda i: tuple(0 for _ in shape))
    return pl.pallas_call(
        _tc_final_body,
        grid=(N // BN,),
        in_specs=[
            pl.BlockSpec((BN, F), lambda i: (i, 0)),
            pl.BlockSpec((1, BN, F), lambda i: (0, i, 0)),
            pl.BlockSpec((1, BN, F), lambda i: (1, i, 0)),
            pl.BlockSpec((BN, 2), lambda i: (i, 0)),
            wspec((F, F)), wspec((1, F)),
            wspec((F, F)), wspec((1, F)),
            wspec((F, F)), wspec((1, F)),
            wspec((3 * F, F)), wspec((1, F)),
            wspec((F, F)), wspec((1, F)),
        ],
        out_specs=pl.BlockSpec((BN, F), lambda i: (i, 0)),
        out_shape=jax.ShapeDtypeStruct((N, F), jnp.float32),
    )(x, q2, q2, dinv, wn, bn, wd, bd, wsl, bsl, wg, bg, wdr, bdr)


def kernel(x, edge_index, edge_type, W_n, b_n, W_d, b_d, W_sl, b_sl, W_gat, b_gat):
    src = edge_index[0].astype(jnp.int32)
    dst = edge_index[1].astype(jnp.int32)
    typ = edge_type.astype(jnp.int32)

    PROBE_SKIP_PREP = False
    if PROBE_SKIP_PREP:
        degp = jnp.zeros((NW, TWO_N), jnp.float32)
        sp0 = jnp.zeros((NW * RSTRIDE,), jnp.int32)
        dp0 = jnp.zeros((NW * RSTRIDE,), jnp.int32)
        sp1 = jnp.zeros((NW * RSTRIDE,), jnp.int32)
        dp1 = jnp.zeros((NW * RSTRIDE,), jnp.int32)
        cnt = jnp.zeros((NW, 16), jnp.int32)
    else:
        degp, sp0, dp0, sp1, dp1, cnt = _sc_prep(src, dst, typ)
    dinv, y0, y1 = _tc_prep(degp.reshape(NW, N, 2), x)
    (q2,) = _sc_agg(y0, y1, sp0, dp0, sp1, dp1, cnt)
    out = _tc_final(
        x, q2, dinv,
        W_n, b_n.reshape(1, F), W_d, b_d.reshape(1, F),
        W_sl, b_sl.reshape(1, F), W_gat, b_gat.reshape(1, F),
        W_d[:, ::-1], b_d[::-1].reshape(1, F),
    )
    return out


# speedup vs baseline: 1.0969x; 1.0678x over previous
import jax
import jax.numpy as jnp
from jax import lax
from jax.experimental import pallas as pl
from jax.experimental.pallas import tpu as pltpu
from jax.experimental.pallas import tpu_sc as plsc

N = 10000
F = 128
E = 320000
TWO_N = 2 * N

NC = 2
NS = 16
NW = NC * NS

NDUM = 8

E_PER_W = E // NW
CH_A = 2000
N_CH_A = E_PER_W // CH_A
RSTRIDE = 10256


def _sc_prep_body(src_hbm, dst_hbm, typ_hbm, degp_out, sp0, dp0, sp1, dp1, cnt_out,
                  src_v, dst_v, typ_v, src_w, dst_w, typ_w,
                  sb0, db0, sb1, db1, cnt_v, deg_v, semp0, semp1):
    c = lax.axis_index("c")
    s = lax.axis_index("s")
    wid = s * NC + c

    def zero_body(j, carry):
        deg_v[pl.ds(pl.multiple_of(j * 16, 16), 16)] = jnp.zeros((16,), jnp.float32)
        return carry

    lax.fori_loop(0, TWO_N // 16, zero_body, 0)

    ones16 = jnp.ones((16,), jnp.float32)
    cur0, cur1 = jnp.int32(0), jnp.int32(0)
    inbufs = ((src_v, dst_v, typ_v), (src_w, dst_w, typ_w))
    semps = (semp0, semp1)

    def in_start(c5, slot):
        base = pl.multiple_of(wid * E_PER_W + c5 * CH_A, 8)
        for hbm, buf in zip((src_hbm, dst_hbm, typ_hbm), inbufs[slot]):
            pltpu.async_copy(hbm.at[pl.ds(base, CH_A)], buf, semps[slot])

    def in_wait(c5, slot):
        base = pl.multiple_of(wid * E_PER_W + c5 * CH_A, 8)
        for hbm, buf in zip((src_hbm, dst_hbm, typ_hbm), inbufs[slot]):
            pltpu.make_async_copy(hbm.at[pl.ds(base, CH_A)], buf,
                                  semps[slot]).wait()

    in_start(0, 0)
    for c5 in range(N_CH_A):
        slot = c5 % 2
        in_wait(c5, slot)
        if c5 + 1 < N_CH_A:
            in_start(c5 + 1, 1 - slot)
        src_c, dst_c, typ_c = inbufs[slot]

        def body(j, carry, src_v=src_c, dst_v=dst_c, typ_v=typ_c):
            c0, c1 = carry
            sl = pl.ds(pl.multiple_of(j * 16, 16), 16)
            s16 = src_v[sl]
            d16 = dst_v[sl]
            t16 = typ_v[sl]
            m0 = t16 == 0
            plsc.store_compressed(sb0.at[pl.ds(c0, 16)], s16, mask=m0)
            plsc.store_compressed(db0.at[pl.ds(c0, 16)], d16, mask=m0)
            m1 = jnp.logical_not(m0)
            plsc.store_compressed(sb1.at[pl.ds(c1, 16)], s16, mask=m1)
            plsc.store_compressed(db1.at[pl.ds(c1, 16)], d16, mask=m1)
            plsc.addupdate_scatter(deg_v, [d16 * 2 + t16], ones16)
            n0 = jnp.sum(jnp.where(m0, 1, 0))
            return (c0 + n0, c1 + (16 - n0))

        cur0, cur1 = lax.fori_loop(0, CH_A // 16, body, (cur0, cur1))

    sfill = jnp.zeros((16,), jnp.int32)
    dfill = jnp.full((16,), N, jnp.int32)
    ftrue = sfill == 0

    def pad_fill(sb, db, cur):
        bound = ((cur + 255) >> 8) << 8
        nf = (bound - cur + 15) >> 4

        def fbody(i, carry):
            plsc.store_compressed(sb.at[pl.ds(cur + i * 16, 16)], sfill, mask=ftrue)
            plsc.store_compressed(db.at[pl.ds(cur + i * 16, 16)], dfill, mask=ftrue)
            return carry

        lax.fori_loop(0, nf, fbody, 0)
        return bound >> 8

    k0 = pad_fill(sb0, db0, cur0)
    k1 = pad_fill(sb1, db1, cur1)
    lane = lax.iota(jnp.int32, 16)
    cnt_v[pl.ds(0, 16)] = jnp.where(lane == 0, k0, jnp.where(lane == 1, k1, 0))

    rb = pl.multiple_of(wid * RSTRIDE, 8)
    pltpu.sync_copy(sb0, sp0.at[pl.ds(rb, RSTRIDE)])
    pltpu.sync_copy(db0, dp0.at[pl.ds(rb, RSTRIDE)])
    pltpu.sync_copy(sb1, sp1.at[pl.ds(rb, RSTRIDE)])
    pltpu.sync_copy(db1, dp1.at[pl.ds(rb, RSTRIDE)])
    pltpu.sync_copy(cnt_v, cnt_out.at[wid])
    pltpu.sync_copy(deg_v, degp_out.at[wid])


@jax.jit
def _sc_prep(src, dst, typ):
    mesh = plsc.VectorSubcoreMesh(core_axis_name="c", subcore_axis_name="s")
    fn = pl.kernel(
        _sc_prep_body,
        mesh=mesh,
        compiler_params=pltpu.CompilerParams(needs_layout_passes=False),
        out_type=[
            jax.ShapeDtypeStruct((NW, TWO_N), jnp.float32),
            jax.ShapeDtypeStruct((NW * RSTRIDE,), jnp.int32),
            jax.ShapeDtypeStruct((NW * RSTRIDE,), jnp.int32),
            jax.ShapeDtypeStruct((NW * RSTRIDE,), jnp.int32),
            jax.ShapeDtypeStruct((NW * RSTRIDE,), jnp.int32),
            jax.ShapeDtypeStruct((NW, 16), jnp.int32),
        ],
        scratch_types=[
            pltpu.VMEM((CH_A,), jnp.int32),
            pltpu.VMEM((CH_A,), jnp.int32),
            pltpu.VMEM((CH_A,), jnp.int32),
            pltpu.VMEM((CH_A,), jnp.int32),
            pltpu.VMEM((CH_A,), jnp.int32),
            pltpu.VMEM((CH_A,), jnp.int32),
            pltpu.VMEM((RSTRIDE,), jnp.int32),
            pltpu.VMEM((RSTRIDE,), jnp.int32),
            pltpu.VMEM((RSTRIDE,), jnp.int32),
            pltpu.VMEM((RSTRIDE,), jnp.int32),
            pltpu.VMEM((16,), jnp.int32),
            pltpu.VMEM((TWO_N,), jnp.float32),
            pltpu.SemaphoreType.DMA,
            pltpu.SemaphoreType.DMA,
        ],
    )
    return fn(src, dst, typ)


BN = 1000


def _tc_prep_body(degp_ref, x_ref, dinv_ref, y0_ref, y1_ref):
    deg = jnp.sum(degp_ref[...], axis=0) + 1.0
    dinv = lax.rsqrt(deg)
    dinv_ref[...] = dinv
    xb = x_ref[...]
    y0_ref[...] = dinv[:, 0:1] * xb
    y1_ref[...] = dinv[:, 1:2] * xb


@jax.jit
def _tc_prep(degp, x):
    return pl.pallas_call(
        _tc_prep_body,
        grid=(N // BN,),
        in_specs=[
            pl.BlockSpec((NW, BN, 2), lambda i: (0, i, 0)),
            pl.BlockSpec((BN, F), lambda i: (i, 0)),
        ],
        out_specs=[
            pl.BlockSpec((BN, 2), lambda i: (i, 0)),
            pl.BlockSpec((BN, F), lambda i: (i, 0)),
            pl.BlockSpec((BN, F), lambda i: (i, 0)),
        ],
        out_shape=[
            jax.ShapeDtypeStruct((N, 2), jnp.float32),
            jax.ShapeDtypeStruct((N, F), jnp.float32),
            jax.ShapeDtypeStruct((N, F), jnp.float32),
        ],
    )(degp, x)


CH_C = 64
NSLOT = 4
ZROWS = 624
ZTAIL = N - NS * ZROWS
ZCH = 48


def _sc_agg_body(y0, y1, sp0, dp0, sp1, dp1, cnt, qout, *scr):
    c = lax.axis_index("c")
    s = lax.axis_index("s")
    sixs = scr[0:NSLOT]
    dixs = scr[NSLOT:2 * NSLOT]
    rows = scr[2 * NSLOT:3 * NSLOT]
    cnt_v, zbuf, q_sh = scr[3 * NSLOT:3 * NSLOT + 3]
    semg = scr[3 * NSLOT + 3:4 * NSLOT + 3]
    sems = scr[4 * NSLOT + 3:5 * NSLOT + 3]
    semi = scr[5 * NSLOT + 3:6 * NSLOT + 3]

    PROBE_SKIP_ZW = False
    z16 = jnp.zeros((16,), jnp.float32)

    def zb_body(r, carry):
        for k in range(F // 16):
            zbuf[r, pl.ds(k * 16, 16)] = z16
        return carry

    lax.fori_loop(0, ZCH, zb_body, 0)
    for kk in range(ZROWS // ZCH if not PROBE_SKIP_ZW else 0):
        zsl = pl.ds(pl.multiple_of(s * ZROWS + kk * ZCH, 8), ZCH)
        pltpu.sync_copy(zbuf, q_sh.at[zsl])

    @pl.when(s < ZTAIL // 8)
    def _():
        tsl = pl.ds(pl.multiple_of(NS * ZROWS + s * 8, 8), 8)
        pltpu.sync_copy(zbuf.at[pl.ds(0, 8)], q_sh.at[tsl])

    @pl.when(s == 2)
    def _():
        dsl = pl.ds(pl.multiple_of(N, 8), NDUM)
        pltpu.sync_copy(zbuf.at[pl.ds(0, NDUM)], q_sh.at[dsl])

    plsc.subcore_barrier()

    def region_loop(ysel, ssel, dsel, tt):
        for rr in range(2):
            r = s * 2 + rr
            pltpu.sync_copy(cnt.at[r], cnt_v)
            kmax2 = cnt_v[pl.ds(0, 16)][tt]
            rbase = r * RSTRIDE

            @pl.when(kmax2 > 0)
            def _():
                nch = kmax2 * NSLOT

                def idx_start(k, slot):
                    bp = pl.multiple_of(rbase + k * CH_C, 8)
                    pltpu.async_copy(ssel.at[pl.ds(bp, CH_C)], sixs[slot],
                                     semi[slot])
                    pltpu.async_copy(dsel.at[pl.ds(bp, CH_C)], dixs[slot],
                                     semi[slot])

                def idx_wait(k, slot):
                    bp = pl.multiple_of(rbase + k * CH_C, 8)
                    pltpu.make_async_copy(ssel.at[pl.ds(bp, CH_C)],
                                          sixs[slot], semi[slot]).wait()
                    pltpu.make_async_copy(dsel.at[pl.ds(bp, CH_C)],
                                          dixs[slot], semi[slot]).wait()

                for p in range(NSLOT - 1):
                    idx_start(p, p)
                    idx_wait(p, p)
                    pltpu.async_copy(ysel.at[sixs[p]], rows[p], semg[p])

                @pl.when(NSLOT - 1 < nch)
                def _():
                    idx_start(NSLOT - 1, NSLOT - 1)

                def group_body(k2, carry):
                    for b in range(NSLOT):
                        o = (b + NSLOT - 1) % NSLOT
                        k = k2 * NSLOT + b

                        pltpu.make_async_copy(
                            ysel.at[sixs[b]], rows[b], semg[b]).wait()
                        pltpu.async_copy(
                            rows[b], q_sh.at[dixs[b]], sems[b], add=True)

                        @pl.when(k >= 1)
                        def _():
                            pltpu.make_async_copy(
                                rows[o], q_sh.at[dixs[o]], sems[o]).wait()

                        @pl.when(k + NSLOT - 1 < nch)
                        def _():
                            idx_wait(k + NSLOT - 1, o)
                            pltpu.async_copy(ysel.at[sixs[o]], rows[o], semg[o])

                        @pl.when(k + NSLOT < nch)
                        def _():
                            idx_start(k + NSLOT, b)
                    return carry

                lax.fori_loop(0, kmax2, group_body, 0)
                pltpu.make_async_copy(
                    rows[NSLOT - 1], q_sh.at[dixs[NSLOT - 1]],
                    sems[NSLOT - 1]).wait()

    PROBE_SKIP_EDGES = False
    if not PROBE_SKIP_EDGES:
        @pl.when(c == 0)
        def _():
            region_loop(y0, sp0, dp0, 0)

        @pl.when(c == 1)
        def _():
            region_loop(y1, sp1, dp1, 1)

    plsc.subcore_barrier()
    for kk in range(ZROWS // ZCH if not PROBE_SKIP_ZW else 1):
        zsl = pl.ds(pl.multiple_of(s * ZROWS + kk * ZCH, 8), ZCH)
        pltpu.sync_copy(q_sh.at[zsl], zbuf)
        pltpu.sync_copy(zbuf, qout.at[c].at[zsl])

    @pl.when(s < ZTAIL // 8)
    def _():
        tsl = pl.ds(pl.multiple_of(NS * ZROWS + s * 8, 8), 8)
        pltpu.sync_copy(q_sh.at[tsl], zbuf.at[pl.ds(0, 8)])
        pltpu.sync_copy(zbuf.at[pl.ds(0, 8)], qout.at[c].at[tsl])


@jax.jit
def _sc_agg(y0, y1, sp0, dp0, sp1, dp1, cnt):
    mesh = plsc.VectorSubcoreMesh(core_axis_name="c", subcore_axis_name="s")
    fn = pl.kernel(
        _sc_agg_body,
        mesh=mesh,
        compiler_params=pltpu.CompilerParams(needs_layout_passes=False),
        out_type=[jax.ShapeDtypeStruct((NC, N, F), jnp.float32)],
        scratch_types=(
            [pltpu.VMEM((CH_C,), jnp.int32)] * (2 * NSLOT)
            + [pltpu.VMEM((CH_C, F), jnp.float32)] * NSLOT
            + [
                pltpu.VMEM((16,), jnp.int32),
                pltpu.VMEM((ZCH, F), jnp.float32),
                pltpu.VMEM_SHARED((N + NDUM, F), jnp.float32),
            ]
            + [pltpu.SemaphoreType.DMA] * (3 * NSLOT)
        ),
    )
    return fn(y0, y1, sp0, dp0, sp1, dp1, cnt)


def _tc_final_body(x_ref, qn_ref, qd_ref, dinv_ref,
                   wn, bn, wd, bd, wsl, bsl, wg, bg, wdr, bdr, out_ref):
    xb = x_ref[...]
    d0 = dinv_ref[:, 0:1]
    d1 = dinv_ref[:, 1:2]
    pre_n = d0 * (qn_ref[0] + d0 * xb)
    pre_d = d1 * (qd_ref[0] + d1 * xb)

    def mm(a, w):
        return jnp.dot(a, w[...], preferred_element_type=jnp.float32)

    xn = mm(pre_n, wn) + bn[...]
    xd = mm(pre_d, wd) + bd[...]
    xx = mm(xb, wsl) + bsl[...]
    wgr = wg[...]
    z = (jnp.dot(xx, wgr[0:F, :], preferred_element_type=jnp.float32)
         + jnp.dot(xn, wgr[F:2 * F, :], preferred_element_type=jnp.float32)
         + jnp.dot(xd, wgr[2 * F:3 * F, :], preferred_element_type=jnp.float32)
         + bg[...])
    m = jnp.max(z, axis=-1, keepdims=True)
    ez = jnp.exp(z - m)
    sm = ez / jnp.sum(ez, axis=-1, keepdims=True)
    rr = lax.broadcasted_iota(jnp.int32, (F, F), 0)
    cc = lax.broadcasted_iota(jnp.int32, (F, F), 1)
    tri = (rr <= cc).astype(jnp.float32)
    gat = jnp.dot(sm, tri, preferred_element_type=jnp.float32)
    xdr = mm(pre_d, wdr) + bdr[...]
    out_ref[...] = xdr * gat + xx + xn


@jax.jit
def _tc_final(x, q2, dinv, wn, bn, wd, bd, wsl, bsl, wg, bg, wdr, bdr):
    wspec = lambda shape: pl.BlockSpec(shape, lambda i: tuple(0 for _ in shape))
    return pl.pallas_call(
        _tc_final_body,
        grid=(N // BN,),
        in_specs=[
            pl.BlockSpec((BN, F), lambda i: (i, 0)),
            pl.BlockSpec((1, BN, F), lambda i: (0, i, 0)),
            pl.BlockSpec((1, BN, F), lambda i: (1, i, 0)),
            pl.BlockSpec((BN, 2), lambda i: (i, 0)),
            wspec((F, F)), wspec((1, F)),
            wspec((F, F)), wspec((1, F)),
            wspec((F, F)), wspec((1, F)),
            wspec((3 * F, F)), wspec((1, F)),
            wspec((F, F)), wspec((1, F)),
        ],
        out_specs=pl.BlockSpec((BN, F), lambda i: (i, 0)),
        out_shape=jax.ShapeDtypeStruct((N, F), jnp.float32),
    )(x, q2, q2, dinv, wn, bn, wd, bd, wsl, bsl, wg, bg, wdr, bdr)


def kernel(x, edge_index, edge_type, W_n, b_n, W_d, b_d, W_sl, b_sl, W_gat, b_gat):
    src = edge_index[0].astype(jnp.int32)
    dst = edge_index[1].astype(jnp.int32)
    typ = edge_type.astype(jnp.int32)

    PROBE_SKIP_PREP = False
    if PROBE_SKIP_PREP:
        degp = jnp.zeros((NW, TWO_N), jnp.float32)
        sp0 = jnp.zeros((NW * RSTRIDE,), jnp.int32)
        dp0 = jnp.zeros((NW * RSTRIDE,), jnp.int32)
        sp1 = jnp.zeros((NW * RSTRIDE,), jnp.int32)
        dp1 = jnp.zeros((NW * RSTRIDE,), jnp.int32)
        cnt = jnp.zeros((NW, 16), jnp.int32)
    else:
        degp, sp0, dp0, sp1, dp1, cnt = _sc_prep(src, dst, typ)
    dinv, y0, y1 = _tc_prep(degp.reshape(NW, N, 2), x)
    (q2,) = _sc_agg(y0, y1, sp0, dp0, sp1, dp1, cnt)
    out = _tc_final(
        x, q2, dinv,
        W_n, b_n.reshape(1, F), W_d, b_d.reshape(1, F),
        W_sl, b_sl.reshape(1, F), W_gat, b_gat.reshape(1, F),
        W_d[:, ::-1], b_d[::-1].reshape(1, F),
    )
    return out


# speedup vs baseline: 1.0974x; 1.0005x over previous
import jax
import jax.numpy as jnp
from jax import lax
from jax.experimental import pallas as pl
from jax.experimental.pallas import tpu as pltpu
from jax.experimental.pallas import tpu_sc as plsc

N = 10000
F = 128
E = 320000
TWO_N = 2 * N

NC = 2
NS = 16
NW = NC * NS

NDUM = 8

E_PER_W = E // NW
CH_A = 2000
N_CH_A = E_PER_W // CH_A
RSTRIDE = 10256


def _sc_prep_body(src_hbm, dst_hbm, typ_hbm, degp_out, sp0, dp0, sp1, dp1, cnt_out,
                  src_v, dst_v, typ_v, src_w, dst_w, typ_w,
                  sb0, db0, sb1, db1, cnt_v, deg_v, semp0, semp1):
    c = lax.axis_index("c")
    s = lax.axis_index("s")
    wid = s * NC + c

    def zero_body(j, carry):
        deg_v[pl.ds(pl.multiple_of(j * 16, 16), 16)] = jnp.zeros((16,), jnp.float32)
        return carry

    lax.fori_loop(0, TWO_N // 16, zero_body, 0)

    ones16 = jnp.ones((16,), jnp.float32)
    cur0, cur1 = jnp.int32(0), jnp.int32(0)
    inbufs = ((src_v, dst_v, typ_v), (src_w, dst_w, typ_w))
    semps = (semp0, semp1)

    def in_start(c5, slot):
        base = pl.multiple_of(wid * E_PER_W + c5 * CH_A, 8)
        for hbm, buf in zip((src_hbm, dst_hbm, typ_hbm), inbufs[slot]):
            pltpu.async_copy(hbm.at[pl.ds(base, CH_A)], buf, semps[slot])

    def in_wait(c5, slot):
        base = pl.multiple_of(wid * E_PER_W + c5 * CH_A, 8)
        for hbm, buf in zip((src_hbm, dst_hbm, typ_hbm), inbufs[slot]):
            pltpu.make_async_copy(hbm.at[pl.ds(base, CH_A)], buf,
                                  semps[slot]).wait()

    in_start(0, 0)
    for c5 in range(N_CH_A):
        slot = c5 % 2
        in_wait(c5, slot)
        if c5 + 1 < N_CH_A:
            in_start(c5 + 1, 1 - slot)
        src_c, dst_c, typ_c = inbufs[slot]

        def body(j, carry, src_v=src_c, dst_v=dst_c, typ_v=typ_c):
            c0, c1 = carry
            sl = pl.ds(pl.multiple_of(j * 16, 16), 16)
            s16 = src_v[sl]
            d16 = dst_v[sl]
            t16 = typ_v[sl]
            m0 = t16 == 0
            plsc.store_compressed(sb0.at[pl.ds(c0, 16)], s16, mask=m0)
            plsc.store_compressed(db0.at[pl.ds(c0, 16)], d16, mask=m0)
            m1 = jnp.logical_not(m0)
            plsc.store_compressed(sb1.at[pl.ds(c1, 16)], s16, mask=m1)
            plsc.store_compressed(db1.at[pl.ds(c1, 16)], d16, mask=m1)
            plsc.addupdate_scatter(deg_v, [d16 * 2 + t16], ones16)
            n0 = jnp.sum(jnp.where(m0, 1, 0))
            return (c0 + n0, c1 + (16 - n0))

        cur0, cur1 = lax.fori_loop(0, CH_A // 16, body, (cur0, cur1))

    sfill = jnp.zeros((16,), jnp.int32)
    dfill = jnp.full((16,), N, jnp.int32)
    ftrue = sfill == 0

    def pad_fill(sb, db, cur):
        bound = ((cur + 255) >> 8) << 8
        nf = (bound - cur + 15) >> 4

        def fbody(i, carry):
            plsc.store_compressed(sb.at[pl.ds(cur + i * 16, 16)], sfill, mask=ftrue)
            plsc.store_compressed(db.at[pl.ds(cur + i * 16, 16)], dfill, mask=ftrue)
            return carry

        lax.fori_loop(0, nf, fbody, 0)
        return bound >> 8

    k0 = pad_fill(sb0, db0, cur0)
    k1 = pad_fill(sb1, db1, cur1)
    lane = lax.iota(jnp.int32, 16)
    cnt_v[pl.ds(0, 16)] = jnp.where(lane == 0, k0, jnp.where(lane == 1, k1, 0))

    rb = pl.multiple_of(wid * RSTRIDE, 8)
    pltpu.sync_copy(sb0, sp0.at[pl.ds(rb, RSTRIDE)])
    pltpu.sync_copy(db0, dp0.at[pl.ds(rb, RSTRIDE)])
    pltpu.sync_copy(sb1, sp1.at[pl.ds(rb, RSTRIDE)])
    pltpu.sync_copy(db1, dp1.at[pl.ds(rb, RSTRIDE)])
    pltpu.sync_copy(cnt_v, cnt_out.at[wid])
    pltpu.sync_copy(deg_v, degp_out.at[wid])


@jax.jit
def _sc_prep(src, dst, typ):
    mesh = plsc.VectorSubcoreMesh(core_axis_name="c", subcore_axis_name="s")
    fn = pl.kernel(
        _sc_prep_body,
        mesh=mesh,
        compiler_params=pltpu.CompilerParams(needs_layout_passes=False),
        out_type=[
            jax.ShapeDtypeStruct((NW, TWO_N), jnp.float32),
            jax.ShapeDtypeStruct((NW * RSTRIDE,), jnp.int32),
            jax.ShapeDtypeStruct((NW * RSTRIDE,), jnp.int32),
            jax.ShapeDtypeStruct((NW * RSTRIDE,), jnp.int32),
            jax.ShapeDtypeStruct((NW * RSTRIDE,), jnp.int32),
            jax.ShapeDtypeStruct((NW, 16), jnp.int32),
        ],
        scratch_types=[
            pltpu.VMEM((CH_A,), jnp.int32),
            pltpu.VMEM((CH_A,), jnp.int32),
            pltpu.VMEM((CH_A,), jnp.int32),
            pltpu.VMEM((CH_A,), jnp.int32),
            pltpu.VMEM((CH_A,), jnp.int32),
            pltpu.VMEM((CH_A,), jnp.int32),
            pltpu.VMEM((RSTRIDE,), jnp.int32),
            pltpu.VMEM((RSTRIDE,), jnp.int32),
            pltpu.VMEM((RSTRIDE,), jnp.int32),
            pltpu.VMEM((RSTRIDE,), jnp.int32),
            pltpu.VMEM((16,), jnp.int32),
            pltpu.VMEM((TWO_N,), jnp.float32),
            pltpu.SemaphoreType.DMA,
            pltpu.SemaphoreType.DMA,
        ],
    )
    return fn(src, dst, typ)


BN = 1000


def _tc_prep_body(degp_ref, x_ref, dinv_ref, y0_ref, y1_ref):
    deg = jnp.sum(degp_ref[...], axis=0) + 1.0
    dinv = lax.rsqrt(deg)
    dinv_ref[...] = dinv
    xb = x_ref[...]
    y0_ref[...] = dinv[:, 0:1] * xb
    y1_ref[...] = dinv[:, 1:2] * xb


@jax.jit
def _tc_prep(degp, x):
    return pl.pallas_call(
        _tc_prep_body,
        grid=(N // BN,),
        in_specs=[
            pl.BlockSpec((NW, BN, 2), lambda i: (0, i, 0)),
            pl.BlockSpec((BN, F), lambda i: (i, 0)),
        ],
        out_specs=[
            pl.BlockSpec((BN, 2), lambda i: (i, 0)),
            pl.BlockSpec((BN, F), lambda i: (i, 0)),
            pl.BlockSpec((BN, F), lambda i: (i, 0)),
        ],
        out_shape=[
            jax.ShapeDtypeStruct((N, 2), jnp.float32),
            jax.ShapeDtypeStruct((N, F), jnp.float32),
            jax.ShapeDtypeStruct((N, F), jnp.float32),
        ],
    )(degp, x)


CH_C = 64
NSLOT = 4
ZROWS = 624
ZTAIL = N - NS * ZROWS
ZCH = 48


def _sc_agg_body(y0, y1, sp0, dp0, sp1, dp1, cnt, qout, *scr):
    c = lax.axis_index("c")
    s = lax.axis_index("s")
    sixs = scr[0:NSLOT]
    dixs = scr[NSLOT:2 * NSLOT]
    rows = scr[2 * NSLOT:3 * NSLOT]
    cnt_v, zbuf, q_sh = scr[3 * NSLOT:3 * NSLOT + 3]
    semg = scr[3 * NSLOT + 3:4 * NSLOT + 3]
    sems = scr[4 * NSLOT + 3:5 * NSLOT + 3]
    semi = scr[5 * NSLOT + 3:6 * NSLOT + 3]

    z16 = jnp.zeros((16,), jnp.float32)

    def zb_body(r, carry):
        for k in range(F // 16):
            zbuf[r, pl.ds(k * 16, 16)] = z16
        return carry

    lax.fori_loop(0, ZCH, zb_body, 0)
    for kk in range(ZROWS // ZCH):
        zsl = pl.ds(pl.multiple_of(s * ZROWS + kk * ZCH, 8), ZCH)
        pltpu.sync_copy(zbuf, q_sh.at[zsl])

    @pl.when(s < ZTAIL // 8)
    def _():
        tsl = pl.ds(pl.multiple_of(NS * ZROWS + s * 8, 8), 8)
        pltpu.sync_copy(zbuf.at[pl.ds(0, 8)], q_sh.at[tsl])

    @pl.when(s == 2)
    def _():
        dsl = pl.ds(pl.multiple_of(N, 8), NDUM)
        pltpu.sync_copy(zbuf.at[pl.ds(0, NDUM)], q_sh.at[dsl])

    plsc.subcore_barrier()

    def region_loop(ysel, ssel, dsel, tt):
        for rr in range(2):
            r = s * 2 + rr
            pltpu.sync_copy(cnt.at[r], cnt_v)
            kmax2 = cnt_v[pl.ds(0, 16)][tt]
            rbase = r * RSTRIDE

            @pl.when(kmax2 > 0)
            def _():
                nch = kmax2 * NSLOT

                def idx_start(k, slot):
                    bp = pl.multiple_of(rbase + k * CH_C, 8)
                    pltpu.async_copy(ssel.at[pl.ds(bp, CH_C)], sixs[slot],
                                     semi[slot])
                    pltpu.async_copy(dsel.at[pl.ds(bp, CH_C)], dixs[slot],
                                     semi[slot])

                def idx_wait(k, slot):
                    bp = pl.multiple_of(rbase + k * CH_C, 8)
                    pltpu.make_async_copy(ssel.at[pl.ds(bp, CH_C)],
                                          sixs[slot], semi[slot]).wait()
                    pltpu.make_async_copy(dsel.at[pl.ds(bp, CH_C)],
                                          dixs[slot], semi[slot]).wait()

                for p in range(NSLOT - 1):
                    idx_start(p, p)
                    idx_wait(p, p)
                    pltpu.async_copy(ysel.at[sixs[p]], rows[p], semg[p])

                @pl.when(NSLOT - 1 < nch)
                def _():
                    idx_start(NSLOT - 1, NSLOT - 1)

                def group_body(k2, carry):
                    for b in range(NSLOT):
                        o = (b + NSLOT - 1) % NSLOT
                        k = k2 * NSLOT + b

                        pltpu.make_async_copy(
                            ysel.at[sixs[b]], rows[b], semg[b]).wait()
                        pltpu.async_copy(
                            rows[b], q_sh.at[dixs[b]], sems[b], add=True)

                        @pl.when(k >= 1)
                        def _():
                            pltpu.make_async_copy(
                                rows[o], q_sh.at[dixs[o]], sems[o]).wait()

                        @pl.when(k + NSLOT - 1 < nch)
                        def _():
                            idx_wait(k + NSLOT - 1, o)
                            pltpu.async_copy(ysel.at[sixs[o]], rows[o], semg[o])

                        @pl.when(k + NSLOT < nch)
                        def _():
                            idx_start(k + NSLOT, b)
                    return carry

                lax.fori_loop(0, kmax2, group_body, 0)
                pltpu.make_async_copy(
                    rows[NSLOT - 1], q_sh.at[dixs[NSLOT - 1]],
                    sems[NSLOT - 1]).wait()

    @pl.when(c == 0)
    def _():
        region_loop(y0, sp0, dp0, 0)

    @pl.when(c == 1)
    def _():
        region_loop(y1, sp1, dp1, 1)

    plsc.subcore_barrier()
    for kk in range(ZROWS // ZCH):
        zsl = pl.ds(pl.multiple_of(s * ZROWS + kk * ZCH, 8), ZCH)
        pltpu.sync_copy(q_sh.at[zsl], zbuf)
        pltpu.sync_copy(zbuf, qout.at[c].at[zsl])

    @pl.when(s < ZTAIL // 8)
    def _():
        tsl = pl.ds(pl.multiple_of(NS * ZROWS + s * 8, 8), 8)
        pltpu.sync_copy(q_sh.at[tsl], zbuf.at[pl.ds(0, 8)])
        pltpu.sync_copy(zbuf.at[pl.ds(0, 8)], qout.at[c].at[tsl])


@jax.jit
def _sc_agg(y0, y1, sp0, dp0, sp1, dp1, cnt):
    mesh = plsc.VectorSubcoreMesh(core_axis_name="c", subcore_axis_name="s")
    fn = pl.kernel(
        _sc_agg_body,
        mesh=mesh,
        compiler_params=pltpu.CompilerParams(needs_layout_passes=False),
        out_type=[jax.ShapeDtypeStruct((NC, N, F), jnp.float32)],
        scratch_types=(
            [pltpu.VMEM((CH_C,), jnp.int32)] * (2 * NSLOT)
            + [pltpu.VMEM((CH_C, F), jnp.float32)] * NSLOT
            + [
                pltpu.VMEM((16,), jnp.int32),
                pltpu.VMEM((ZCH, F), jnp.float32),
                pltpu.VMEM_SHARED((N + NDUM, F), jnp.float32),
            ]
            + [pltpu.SemaphoreType.DMA] * (3 * NSLOT)
        ),
    )
    return fn(y0, y1, sp0, dp0, sp1, dp1, cnt)


def _tc_final_body(x_ref, qn_ref, qd_ref, dinv_ref,
                   wn, bn, wd, bd, wsl, bsl, wg, bg, wdr, bdr, out_ref):
    xb = x_ref[...]
    d0 = dinv_ref[:, 0:1]
    d1 = dinv_ref[:, 1:2]
    pre_n = d0 * (qn_ref[0] + d0 * xb)
    pre_d = d1 * (qd_ref[0] + d1 * xb)

    def mm(a, w):
        return jnp.dot(a, w[...], preferred_element_type=jnp.float32)

    xn = mm(pre_n, wn) + bn[...]
    xd = mm(pre_d, wd) + bd[...]
    xx = mm(xb, wsl) + bsl[...]
    wgr = wg[...]
    z = (jnp.dot(xx, wgr[0:F, :], preferred_element_type=jnp.float32)
         + jnp.dot(xn, wgr[F:2 * F, :], preferred_element_type=jnp.float32)
         + jnp.dot(xd, wgr[2 * F:3 * F, :], preferred_element_type=jnp.float32)
         + bg[...])
    m = jnp.max(z, axis=-1, keepdims=True)
    ez = jnp.exp(z - m)
    sm = ez / jnp.sum(ez, axis=-1, keepdims=True)
    rr = lax.broadcasted_iota(jnp.int32, (F, F), 0)
    cc = lax.broadcasted_iota(jnp.int32, (F, F), 1)
    tri = (rr <= cc).astype(jnp.float32)
    gat = jnp.dot(sm, tri, preferred_element_type=jnp.float32)
    xdr = mm(pre_d, wdr) + bdr[...]
    out_ref[...] = xdr * gat + xx + xn


@jax.jit
def _tc_final(x, q2, dinv, wn, bn, wd, bd, wsl, bsl, wg, bg, wdr, bdr):
    wspec = lambda shape: pl.BlockSpec(shape, lambda i: tuple(0 for _ in shape))
    return pl.pallas_call(
        _tc_final_body,
        grid=(N // BN,),
        in_specs=[
            pl.BlockSpec((BN, F), lambda i: (i, 0)),
            pl.BlockSpec((1, BN, F), lambda i: (0, i, 0)),
            pl.BlockSpec((1, BN, F), lambda i: (1, i, 0)),
            pl.BlockSpec((BN, 2), lambda i: (i, 0)),
            wspec((F, F)), wspec((1, F)),
            wspec((F, F)), wspec((1, F)),
            wspec((F, F)), wspec((1, F)),
            wspec((3 * F, F)), wspec((1, F)),
            wspec((F, F)), wspec((1, F)),
        ],
        out_specs=pl.BlockSpec((BN, F), lambda i: (i, 0)),
        out_shape=jax.ShapeDtypeStruct((N, F), jnp.float32),
    )(x, q2, q2, dinv, wn, bn, wd, bd, wsl, bsl, wg, bg, wdr, bdr)


def kernel(x, edge_index, edge_type, W_n, b_n, W_d, b_d, W_sl, b_sl, W_gat, b_gat):
    src = edge_index[0].astype(jnp.int32)
    dst = edge_index[1].astype(jnp.int32)
    typ = edge_type.astype(jnp.int32)

    degp, sp0, dp0, sp1, dp1, cnt = _sc_prep(src, dst, typ)
    dinv, y0, y1 = _tc_prep(degp.reshape(NW, N, 2), x)
    (q2,) = _sc_agg(y0, y1, sp0, dp0, sp1, dp1, cnt)
    out = _tc_final(
        x, q2, dinv,
        W_n, b_n.reshape(1, F), W_d, b_d.reshape(1, F),
        W_sl, b_sl.reshape(1, F), W_gat, b_gat.reshape(1, F),
        W_d[:, ::-1], b_d[::-1].reshape(1, F),
    )
    return out
